# baseline (speedup 1.0000x reference)
import jax
import jax.numpy as jnp
from jax.experimental import pallas as pl


def _add_pe_kernel(seq_ref, pe_ref, out_ref):
    out_ref[0] = seq_ref[0] + pe_ref[...]


def kernel(sequence, pe):
    B, S, D = sequence.shape
    BS = 512
    while S % BS:
        BS //= 2
    grid = (S // BS, B)
    return pl.pallas_call(
        _add_pe_kernel,
        grid=grid,
        in_specs=[
            pl.BlockSpec((1, BS, D), lambda i, j: (j, i, 0)),
            pl.BlockSpec((BS, D), lambda i, j: (i, 0)),
        ],
        out_specs=pl.BlockSpec((1, BS, D), lambda i, j: (j, i, 0)),
        out_shape=jax.ShapeDtypeStruct((B, S, D), sequence.dtype),
    )(sequence, pe[:S])


# speedup vs baseline: 1.1151x; 1.1151x over previous
import jax
import jax.numpy as jnp
from jax.experimental import pallas as pl


def _add_pe_kernel(seq_ref, pe_ref, out_ref):
    out_ref[0] = seq_ref[0] + pe_ref[...]


def kernel(sequence, pe):
    B, S, D = sequence.shape
    BS = 1024
    while S % BS:
        BS //= 2
    grid = (S // BS, B)
    return pl.pallas_call(
        _add_pe_kernel,
        grid=grid,
        in_specs=[
            pl.BlockSpec((1, BS, D), lambda i, j: (j, i, 0)),
            pl.BlockSpec((BS, D), lambda i, j: (i, 0)),
        ],
        out_specs=pl.BlockSpec((1, BS, D), lambda i, j: (j, i, 0)),
        out_shape=jax.ShapeDtypeStruct((B, S, D), sequence.dtype),
    )(sequence, pe[:S])


# speedup vs baseline: 1.1635x; 1.0434x over previous
import jax
import jax.numpy as jnp
from jax.experimental import pallas as pl


def _add_pe_kernel(seq_ref, pe_ref, out_ref):
    out_ref[0] = seq_ref[0] + pe_ref[...]


def kernel(sequence, pe):
    B, S, D = sequence.shape
    BS = 2048
    while S % BS:
        BS //= 2
    grid = (S // BS, B)
    return pl.pallas_call(
        _add_pe_kernel,
        grid=grid,
        in_specs=[
            pl.BlockSpec((1, BS, D), lambda i, j: (j, i, 0)),
            pl.BlockSpec((BS, D), lambda i, j: (i, 0)),
        ],
        out_specs=pl.BlockSpec((1, BS, D), lambda i, j: (j, i, 0)),
        out_shape=jax.ShapeDtypeStruct((B, S, D), sequence.dtype),
    )(sequence, pe[:S])
